# baseline (speedup 1.0000x reference)
import functools
import math

import jax
import jax.numpy as jnp
from jax import lax
from jax.experimental import pallas as pl
from jax.experimental.pallas import tpu as pltpu
from jax.experimental.pallas import tpu_sc as plsc

_MARGIN = 0.2
_TAU = 0.1
_K = 8
_THRESH = 100.0
_NEG = -1e30


def _main_kernel(emb_ref, lab_ref, xm2_ref, rowdata_ref, scal_ref):
    f32 = jnp.float32
    emb = emb_ref[...]
    labf = lab_ref[...]
    B = emb.shape[0]

    thr = _K * _TAU * math.log(_THRESH)

    ohc = lax.broadcasted_iota(jnp.int32, (B, 128), 1).astype(f32)
    onehot = jnp.where(labf == ohc, 1.0, 0.0)
    eq = lax.dot_general(
        onehot, onehot, (((1,), (1,)), ((), ())),
        preferred_element_type=f32, precision=lax.Precision.HIGHEST)

    score = lax.dot_general(
        emb, emb, (((1,), (1,)), ((), ())),
        preferred_element_type=f32, precision=lax.Precision.HIGHEST)
    xh = score + _MARGIN * (1.0 - eq)

    row_i = lax.broadcasted_iota(jnp.int32, (B, B), 0)
    col_i = lax.broadcasted_iota(jnp.int32, (B, B), 1)
    offdiag = row_i != col_i
    pos = jnp.logical_and(offdiag, eq > 0.5)

    cnt = jnp.sum(eq, axis=1, keepdims=True)
    active = cnt > 1.0
    kf = jnp.asarray(float(_K), f32)

    xm1 = jnp.where(offdiag, xh, _NEG)
    work = xm1
    tops = []
    for t in range(_K + 1):
        m = jnp.max(work, axis=1, keepdims=True)
        tops.append(m)
        if t < _K:
            work = jnp.where(work == m, _NEG, work)
    fhk1 = sum(tops[:_K])
    h1 = (tops[_K - 1] - tops[_K] >= thr).astype(f32)
    s1 = 1.0 - h1

    xm2 = jnp.where(pos, xh, _NEG)
    xm2_ref[...] = xm2
    predsum = jnp.sum(jnp.where(xm2 >= tops[_K - 1], 1.0, 0.0),
                      axis=1, keepdims=True)
    possum = jnp.sum(jnp.where(xm2 > -1e29, xm2, 0.0), axis=1, keepdims=True)

    log2e = 1.4426950408889634
    c0 = log2e / _TAU

    def soft_topk(masked_x, kvec=None, rounds=_K):
        l = masked_x * c0
        total = jnp.zeros((B, 1), f32)
        for it in range(rounds):
            m = jnp.max(l, axis=1, keepdims=True)
            u = jnp.exp2(l - m)
            z = jnp.sum(u, axis=1, keepdims=True)
            s = jnp.sum(u * xh, axis=1, keepdims=True)
            step = s / z
            if kvec is None:
                total = total + step
            else:
                total = total + jnp.where(float(it) < kvec, step, 0.0)
            if it < rounds - 1:
                l = l + jnp.log(jnp.maximum(z - u, z * 1e-6)) * log2e
        return total

    fsk1 = soft_topk(xm1)
    fsk2 = jnp.where(active, soft_topk(xm2), 0.0)
    kvec = kf - (cnt - 1.0)
    xm3 = jnp.where(eq > 0.5, _NEG, xm1)
    fskneg = soft_topk(xm3, kvec=kvec, rounds=_K - 1)

    small = jnp.logical_and(active, cnt - 1.0 < kf)
    b = fsk1 - possum - fskneg
    loss2 = jnp.sum(jnp.where(small, b / cnt, 0.0))

    real_gt = jnp.minimum(kf, cnt - 1.0)
    err_pos = jnp.sum(jnp.where(active, real_gt - predsum, 0.0))

    D = emb.shape[1]
    gram = lax.dot_general(
        emb, emb, (((0,), (0,)), ((), ())),
        preferred_element_type=f32, precision=lax.Precision.HIGHEST)
    mu = jnp.mean(emb, axis=0, keepdims=True)
    outer = lax.dot_general(
        mu, mu, (((0,), (0,)), ((), ())),
        preferred_element_type=f32, precision=lax.Precision.HIGHEST)
    ri = lax.broadcasted_iota(jnp.int32, (D, D), 0)
    ci = lax.broadcasted_iota(jnp.int32, (D, D), 1)
    eye = jnp.where(ri == ci, 1.0, 0.0)
    cm = gram * (1.0 / B) - outer - eye
    loss3 = jnp.sqrt(jnp.sum(cm * cm))

    v1 = fsk1 * s1 + fhk1 * h1
    rowdata_ref[...] = jnp.concatenate(
        [v1, fsk2, cnt, jnp.zeros((B, 1), f32)], axis=1)
    scal_ref[0] = loss2 + 0.1 * loss3
    scal_ref[1] = err_pos


def _sc_topk2(xm2_hbm, out_hbm, rows_v, res_v, sem):
    NC = 2
    wid = lax.axis_index("s") * NC + lax.axis_index("c")
    base = wid * 16
    thr = _K * _TAU * math.log(_THRESH)
    pltpu.sync_copy(xm2_hbm.at[pl.ds(base, 16), :], rows_v)

    gdn = lax.GatherDimensionNumbers(
        offset_dims=(), collapsed_slice_dims=(0,), start_index_map=(0,))

    def lane_shuffle(v, s):
        idx = jnp.bitwise_xor(lax.iota(jnp.int32, 16), s).reshape(16, 1)
        return lax.gather(v, idx, gdn, slice_sizes=(1,),
                          mode=lax.GatherScatterMode.PROMISE_IN_BOUNDS)

    def bcast_max(v):
        for s in (1, 2, 4, 8):
            v = jnp.maximum(v, lane_shuffle(v, s))
        return v

    def row_body(r, carry):
        idx16 = lax.iota(jnp.int32, 16)
        vm = jnp.full((16,), _NEG, jnp.float32)
        for c in range(32):
            v = rows_v[r, pl.ds(16 * c, 16)]
            vm = jnp.maximum(vm, v)
        tops_vec = jnp.zeros((16,), jnp.float32)
        for t in range(_K + 1):
            m = bcast_max(vm)
            tops_vec = jnp.where(idx16 == t, m, tops_vec)
            if t < _K:
                vm2 = jnp.full((16,), _NEG, jnp.float32)
                for c in range(32):
                    v = rows_v[r, pl.ds(16 * c, 16)]
                    v = jnp.where(v == m, _NEG, v)
                    rows_v[r, pl.ds(16 * c, 16)] = v
                    vm2 = jnp.maximum(vm2, v)
                vm = vm2
        res_v[r, :] = tops_vec
        return carry

    lax.fori_loop(0, 16, row_body, 0)
    pltpu.sync_copy(res_v, out_hbm.at[pl.ds(base, 16), :])


def _combine_kernel(rd_ref, sc_ref, scal_ref, out_ref):
    f32 = jnp.float32
    rd = rd_ref[...]
    scr = sc_ref[...]
    v1 = rd[:, 0:1]
    fsk2 = rd[:, 1:2]
    cnt = rd[:, 2:3]
    thr = _K * _TAU * math.log(_THRESH)
    fhk2raw = jnp.sum(scr[:, 0:_K], axis=1, keepdims=True)
    h2raw = (scr[:, _K - 1:_K] - scr[:, _K:_K + 1] >= thr).astype(f32)
    kf = jnp.asarray(float(_K), f32)
    active = cnt > 1.0
    h2 = jnp.where(cnt - 1.0 < kf + 1.0, 1.0, h2raw)
    s2 = 1.0 - h2
    fhk2 = jnp.where(cnt - 1.0 >= kf, fhk2raw, 0.0)
    a = v1 - fsk2 * s2 - fhk2 * h2
    big = jnp.logical_and(active, cnt - 1.0 >= kf)
    loss1 = jnp.sum(jnp.where(big, a / cnt, 0.0))
    out_ref[0] = loss1 + scal_ref[0]
    out_ref[1] = scal_ref[1]
    out_ref[2] = 0.0


@jax.jit
def kernel(embedding, label):
    B = embedding.shape[0]
    labf = label.astype(jnp.float32).reshape(B, 1)
    xm2, rowdata, scal = pl.pallas_call(
        _main_kernel,
        out_shape=[
            jax.ShapeDtypeStruct((B, B), jnp.float32),
            jax.ShapeDtypeStruct((B, 4), jnp.float32),
            jax.ShapeDtypeStruct((4,), jnp.float32),
        ],
        out_specs=[
            pl.BlockSpec(memory_space=pltpu.VMEM),
            pl.BlockSpec(memory_space=pltpu.VMEM),
            pl.BlockSpec(memory_space=pltpu.SMEM),
        ],
    )(embedding, labf)

    mesh = plsc.VectorSubcoreMesh(core_axis_name="c", subcore_axis_name="s")
    sc_out = functools.partial(
        pl.kernel,
        mesh=mesh,
        out_type=jax.ShapeDtypeStruct((B, 16), jnp.float32),
        scratch_types=[
            pltpu.VMEM((16, B), jnp.float32),
            pltpu.VMEM((16, 16), jnp.float32),
            pltpu.SemaphoreType.DMA,
        ],
    )(_sc_topk2)(xm2)

    out = pl.pallas_call(
        _combine_kernel,
        out_shape=jax.ShapeDtypeStruct((4,), jnp.float32),
        in_specs=[
            pl.BlockSpec(memory_space=pltpu.VMEM),
            pl.BlockSpec(memory_space=pltpu.VMEM),
            pl.BlockSpec(memory_space=pltpu.SMEM),
        ],
        out_specs=pl.BlockSpec(memory_space=pltpu.SMEM),
    )(rowdata, sc_out, scal)
    return (out[0], out[1], out[2])


# speedup vs baseline: 2.2010x; 2.2010x over previous
import math

import jax
import jax.numpy as jnp
from jax import lax
from jax.experimental import pallas as pl
from jax.experimental.pallas import tpu as pltpu

_MARGIN = 0.2
_TAU = 0.1
_K = 8
_THRESH = 100.0
_NEG = -1e30


def _loss_kernel(emb_ref, lab_ref, out_ref):
    f32 = jnp.float32
    emb = emb_ref[...]
    labf = lab_ref[...]
    B = emb.shape[0]

    thr = _K * _TAU * math.log(_THRESH)

    ohc = lax.broadcasted_iota(jnp.int32, (B, 128), 1).astype(f32)
    onehot = jnp.where(labf == ohc, 1.0, 0.0)
    eq = lax.dot_general(
        onehot, onehot, (((1,), (1,)), ((), ())),
        preferred_element_type=f32, precision=lax.Precision.HIGHEST)

    score = lax.dot_general(
        emb, emb, (((1,), (1,)), ((), ())),
        preferred_element_type=f32, precision=lax.Precision.HIGHEST)
    xh = score + _MARGIN * (1.0 - eq)

    row_i = lax.broadcasted_iota(jnp.int32, (B, B), 0)
    col_i = lax.broadcasted_iota(jnp.int32, (B, B), 1)
    offdiag = row_i != col_i
    pos = jnp.logical_and(offdiag, eq > 0.5)

    cnt = jnp.sum(eq, axis=1, keepdims=True)
    active = cnt > 1.0
    kf = jnp.asarray(float(_K), f32)

    xm1 = jnp.where(offdiag, xh, _NEG)
    work = xm1
    tops = []
    for t in range(_K + 1):
        m = jnp.max(work, axis=1, keepdims=True)
        tops.append(m)
        if t < _K:
            work = jnp.where(work == m, _NEG, work)
    fhk1 = sum(tops[:_K])
    h1 = (tops[_K - 1] - tops[_K] >= thr).astype(f32)
    s1 = 1.0 - h1

    xm2 = jnp.where(pos, xh, _NEG)
    predsum = jnp.sum(jnp.where(xm2 >= tops[_K - 1], 1.0, 0.0),
                      axis=1, keepdims=True)
    work = xm2
    tops2 = []
    for t in range(_K + 1):
        m = jnp.max(work, axis=1, keepdims=True)
        tops2.append(m)
        if t < _K:
            work = jnp.where(work == m, _NEG, work)
    fhk2raw = sum(tops2[:_K])
    h2raw = (tops2[_K - 1] - tops2[_K] >= thr).astype(f32)
    h2 = jnp.where(cnt - 1.0 < kf + 1.0, 1.0, h2raw)
    s2 = 1.0 - h2
    fhk2 = jnp.where(cnt - 1.0 >= kf, fhk2raw, 0.0)

    possum = jnp.sum(jnp.where(xm2 > -1e29, xm2, 0.0), axis=1, keepdims=True)

    log2e = 1.4426950408889634
    c0 = log2e / _TAU

    def soft_topk(masked_x, kvec=None, rounds=_K):
        l = masked_x * c0
        total = jnp.zeros((B, 1), f32)
        for it in range(rounds):
            m = jnp.max(l, axis=1, keepdims=True)
            u = jnp.exp2(l - m)
            z = jnp.sum(u, axis=1, keepdims=True)
            s = jnp.sum(u * xh, axis=1, keepdims=True)
            step = s / z
            if kvec is None:
                total = total + step
            else:
                total = total + jnp.where(float(it) < kvec, step, 0.0)
            if it < rounds - 1:
                l = l + jnp.log(jnp.maximum(z - u, z * 1e-6)) * log2e
        return total

    fsk1 = soft_topk(xm1)
    fsk2 = jnp.where(active, soft_topk(xm2), 0.0)
    kvec = kf - (cnt - 1.0)
    xm3 = jnp.where(eq > 0.5, _NEG, xm1)
    fskneg = soft_topk(xm3, kvec=kvec, rounds=_K - 1)

    big = jnp.logical_and(active, cnt - 1.0 >= kf)
    small = jnp.logical_and(active, cnt - 1.0 < kf)
    a = fsk1 * s1 + fhk1 * h1 - fsk2 * s2 - fhk2 * h2
    loss1 = jnp.sum(jnp.where(big, a / cnt, 0.0))
    b = fsk1 - possum - fskneg
    loss2 = jnp.sum(jnp.where(small, b / cnt, 0.0))

    real_gt = jnp.minimum(kf, cnt - 1.0)
    err_pos = jnp.sum(jnp.where(active, real_gt - predsum, 0.0))

    D = emb.shape[1]
    gram = lax.dot_general(
        emb, emb, (((0,), (0,)), ((), ())),
        preferred_element_type=f32, precision=lax.Precision.HIGHEST)
    mu = jnp.mean(emb, axis=0, keepdims=True)
    outer = lax.dot_general(
        mu, mu, (((0,), (0,)), ((), ())),
        preferred_element_type=f32, precision=lax.Precision.HIGHEST)
    ri = lax.broadcasted_iota(jnp.int32, (D, D), 0)
    ci = lax.broadcasted_iota(jnp.int32, (D, D), 1)
    eye = jnp.where(ri == ci, 1.0, 0.0)
    cm = gram * (1.0 / B) - outer - eye
    loss3 = jnp.sqrt(jnp.sum(cm * cm))

    loss = loss1 + loss2 + 0.1 * loss3

    out_ref[0] = loss
    out_ref[1] = err_pos
    out_ref[2] = 0.0


@jax.jit
def kernel(embedding, label):
    labf = label.astype(jnp.float32).reshape(label.shape[0], 1)
    out = pl.pallas_call(
        _loss_kernel,
        out_shape=jax.ShapeDtypeStruct((4,), jnp.float32),
        out_specs=pl.BlockSpec(memory_space=pltpu.SMEM),
    )(embedding, labf)
    return (out[0], out[1], out[2])
